# Initial kernel scaffold; baseline (speedup 1.0000x reference)
import jax
import jax.numpy as jnp
from jax.experimental import pallas as pl


def kernel(h_feats, x, edge_index, spatial_attr, positional_attr, W_single, W_spatial, W_pos, ew1, eb1, ew2, eb2, aw, ab, nw1, nb1, nw2, nb2):
    raise NotImplementedError("write your pallas kernel here")


# speedup vs baseline: 3.0997x; 3.0997x over previous
import functools

import jax
import jax.numpy as jnp
from jax import lax
from jax.experimental import pallas as pl
from jax.experimental.pallas import tpu as pltpu

try:
    from jax.experimental.pallas import tpu_sc as plsc
    _HAS_SC = True
except ImportError:
    plsc = None
    _HAS_SC = False

_F32 = jnp.float32

_NC = 2
_NS = 16
_NW = _NC * _NS
_CH = 128


def _node_block(n):
    for b in (400, 200, 100, 40, 8):
        if n % b == 0:
            return b
    return n


def _edge_block(e):
    for b in (2000, 1000, 400, 200, 40, 8):
        if e % b == 0:
            return b
    return e


def _embed_body(hf, ws, wa, wb, h_o, p_o, q_o):
    h = jnp.dot(hf[...], ws[...], preferred_element_type=_F32)
    h_o[...] = h
    p_o[...] = jnp.dot(h, wa[...], preferred_element_type=_F32)
    q_o[...] = jnp.dot(h, wb[...], preferred_element_type=_F32)


def _embed_call(h_feats, W_single, wa, wb):
    n, lm = h_feats.shape
    nd = W_single.shape[1]
    bn = _node_block(n)
    grid = (n // bn,)
    full = lambda s: pl.BlockSpec(s, lambda i: (0, 0))
    row = lambda c: pl.BlockSpec((bn, c), lambda i: (i, 0))
    out_sd = jax.ShapeDtypeStruct((n, nd), _F32)
    return pl.pallas_call(
        _embed_body,
        grid=grid,
        in_specs=[row(lm), full((lm, nd)), full((nd, nd)), full((nd, nd))],
        out_specs=[row(nd), row(nd), row(nd)],
        out_shape=[out_sd, out_sd, out_sd],
    )(h_feats, W_single, wa, wb)


def _eemb_body(sp, po, wsp, wpo, ea_o):
    ea = jnp.dot(sp[...], wsp[...], preferred_element_type=_F32)
    ea += jnp.dot(po[...], wpo[...], preferred_element_type=_F32)
    ea_o[...] = ea


def _eemb_call(spatial, pos, wsp, wpo):
    e, spd = spatial.shape
    ped = pos.shape[1]
    ed = wsp.shape[1]
    be = _edge_block(e)
    grid = (e // be,)
    full = lambda s: pl.BlockSpec(s, lambda i: (0, 0))
    row = lambda c: pl.BlockSpec((be, c), lambda i: (i, 0))
    return pl.pallas_call(
        _eemb_body,
        grid=grid,
        in_specs=[row(spd), row(ped), full((spd, ed)), full((ped, ed))],
        out_specs=row(ed),
        out_shape=jax.ShapeDtypeStruct((e, ed), _F32),
    )(spatial, pos, wsp, wpo)


def _edge_body(ps, qd, ea, rad, w1e, w1r, b1, w2, b2, awv, abv, mo_o):
    t = ps[...] + qd[...] + rad[...] * w1r[...] + b1[...]
    t += jnp.dot(ea[...], w1e[...], preferred_element_type=_F32)
    t = t * jax.nn.sigmoid(t)
    m = jnp.dot(t, w2[...], preferred_element_type=_F32) + b2[...]
    m = m * jax.nn.sigmoid(m)
    lg = jnp.sum(m * awv[...], axis=1, keepdims=True) + abv[...]
    mo_o[...] = m * jax.nn.sigmoid(lg)


def _edge_call(psrc, qdst, eattr, rad, w1e, w1r, b1, w2, b2, awv, abv):
    e, nd = psrc.shape
    ed = eattr.shape[1]
    be = _edge_block(e)
    grid = (e // be,)
    full = lambda s: pl.BlockSpec(s, lambda i: tuple(0 for _ in s))
    row = lambda c: pl.BlockSpec((be, c), lambda i: (i, 0))
    return pl.pallas_call(
        _edge_body,
        grid=grid,
        in_specs=[row(nd), row(nd), row(ed), row(1), full((ed, nd)),
                  full((1, nd)), full((1, nd)), full((nd, nd)),
                  full((1, nd)), full((1, nd)), full((1, 1))],
        out_specs=row(nd),
        out_shape=jax.ShapeDtypeStruct((e, nd), _F32),
    )(psrc, qdst, eattr, rad, w1e, w1r, b1, w2, b2, awv, abv)


def _node_body_pq(h, a0, a1, w1h, w1a, b1, w2, b2, wan, wbn, h_o, p_o, q_o):
    agg = a0[...] + a1[...]
    o = jnp.dot(h[...], w1h[...], preferred_element_type=_F32)
    o += jnp.dot(agg, w1a[...], preferred_element_type=_F32) + b1[...]
    o = o * jax.nn.sigmoid(o)
    o = jnp.dot(o, w2[...], preferred_element_type=_F32) + b2[...]
    hn = h[...] + o
    h_o[...] = hn
    p_o[...] = jnp.dot(hn, wan[...], preferred_element_type=_F32)
    q_o[...] = jnp.dot(hn, wbn[...], preferred_element_type=_F32)


def _node_body_last(h, a0, a1, w1h, w1a, b1, w2, b2, h_o):
    agg = a0[...] + a1[...]
    o = jnp.dot(h[...], w1h[...], preferred_element_type=_F32)
    o += jnp.dot(agg, w1a[...], preferred_element_type=_F32) + b1[...]
    o = o * jax.nn.sigmoid(o)
    o = jnp.dot(o, w2[...], preferred_element_type=_F32) + b2[...]
    h_o[...] = h[...] + o


def _node_call(h, a0, a1, w1h, w1a, b1, w2, b2, wan=None, wbn=None):
    n, nd = h.shape
    bn = _node_block(n)
    grid = (n // bn,)
    full = lambda s: pl.BlockSpec(s, lambda i: (0, 0))
    row = pl.BlockSpec((bn, nd), lambda i: (i, 0))
    out_sd = jax.ShapeDtypeStruct((n, nd), _F32)
    wspecs = [full((nd, nd)), full((nd, nd)), full((1, nd)), full((nd, nd)),
              full((1, nd))]
    if wan is None:
        return pl.pallas_call(
            _node_body_last,
            grid=grid,
            in_specs=[row, row, row] + wspecs,
            out_specs=row,
            out_shape=out_sd,
        )(h, a0, a1, w1h, w1a, b1, w2, b2)
    return pl.pallas_call(
        _node_body_pq,
        grid=grid,
        in_specs=[row, row, row] + wspecs + [full((nd, nd)), full((nd, nd))],
        out_specs=[row, row, row],
        out_shape=[out_sd, out_sd, out_sd],
    )(h, a0, a1, w1h, w1a, b1, w2, b2, wan, wbn)


def _gather_pq_sc(p_tab, q_tab, src, dst, xcols=None):
    n, nd = p_tab.shape
    e = src.shape[0]
    nchunks = e // _CH
    with_rad = xcols is not None
    mesh = plsc.VectorSubcoreMesh(core_axis_name="c", subcore_axis_name="s")
    out_type = [jax.ShapeDtypeStruct((e, nd), _F32),
                jax.ShapeDtypeStruct((e, nd), _F32)]
    scratch = [pltpu.VMEM((_CH,), jnp.int32), pltpu.VMEM((_CH,), jnp.int32),
               pltpu.VMEM((_CH, nd), _F32), pltpu.VMEM((_CH, nd), _F32)]
    if with_rad:
        out_type.append(jax.ShapeDtypeStruct((e,), _F32))
        scratch += [pltpu.VMEM((n,), _F32)] * 3 + [pltpu.VMEM((_CH,), _F32)]
    scratch += [pltpu.SemaphoreType.DMA]

    @functools.partial(
        pl.kernel, mesh=mesh, out_type=out_type, scratch_types=scratch,
        compiler_params=pltpu.CompilerParams(needs_layout_passes=False))
    def gk(*refs):
        nin = 4 + (3 if with_rad else 0)
        nout = 3 if with_rad else 2
        p_h, q_h = refs[0], refs[1]
        xc_h = refs[2:2 + (3 if with_rad else 0)]
        src_h, dst_h = refs[nin - 2], refs[nin - 1]
        outs = refs[nin:nin + nout]
        idx_s, idx_d, pbuf, qbuf = refs[nin + nout:nin + nout + 4]
        xv = refs[nin + nout + 4:nin + nout + 7] if with_rad else ()
        rbuf = refs[nin + nout + 7] if with_rad else None
        sem = refs[-1]
        if with_rad:
            for k in range(3):
                pltpu.sync_copy(xc_h[k], xv[k])
        wid = lax.axis_index("s") * _NC + lax.axis_index("c")
        extra = nchunks - (nchunks // _NW) * _NW
        nit = (nchunks // _NW) + jnp.where(wid < extra, 1, 0)

        def body(j, carry):
            base = (wid + j * _NW) * _CH
            pltpu.sync_copy(src_h.at[pl.ds(base, _CH)], idx_s)
            pltpu.sync_copy(dst_h.at[pl.ds(base, _CH)], idx_d)
            cp1 = pltpu.async_copy(p_h.at[idx_s], pbuf, sem)
            cp2 = pltpu.async_copy(q_h.at[idx_d], qbuf, sem)
            if with_rad:
                for i in range(_CH // 16):
                    s_ids = idx_s[pl.ds(i * 16, 16)]
                    d_ids = idx_d[pl.ds(i * 16, 16)]
                    acc = None
                    for k in range(3):
                        dd = (plsc.load_gather(xv[k], [s_ids])
                              - plsc.load_gather(xv[k], [d_ids]))
                        acc = dd * dd if acc is None else acc + dd * dd
                    rbuf[pl.ds(i * 16, 16)] = acc
            cp1.wait()
            cp2.wait()
            pltpu.sync_copy(pbuf, outs[0].at[pl.ds(base, _CH)])
            pltpu.sync_copy(qbuf, outs[1].at[pl.ds(base, _CH)])
            if with_rad:
                pltpu.sync_copy(rbuf, outs[2].at[pl.ds(base, _CH)])
            return carry

        lax.fori_loop(0, nit, body, 0)

    if with_rad:
        return gk(p_tab, q_tab, *xcols, src, dst)
    return gk(p_tab, q_tab, src, dst)


def _scatter_add_sc(mo, src, zeros_n):
    e, nd = mo.shape
    n = zeros_n.shape[0]
    e_half = e // _NC
    nchunks = e_half // _CH
    rows_per_sub = n // _NS
    mesh = plsc.VectorSubcoreMesh(core_axis_name="c", subcore_axis_name="s")

    @functools.partial(
        pl.kernel, mesh=mesh,
        out_type=jax.ShapeDtypeStruct((_NC, n, nd), _F32),
        scratch_types=[
            pltpu.VMEM((_CH,), jnp.int32),
            pltpu.VMEM((_CH, nd), _F32),
            pltpu.VMEM_SHARED((n, nd), _F32),
            pltpu.SemaphoreType.DMA,
        ])
    def sk(mo_h, src_h, z_h, out_h, idx_b, m_b, agg_sh, sem):
        cid = lax.axis_index("c")
        sid = lax.axis_index("s")
        rs = sid * rows_per_sub
        pltpu.sync_copy(z_h.at[pl.ds(rs, rows_per_sub)],
                        agg_sh.at[pl.ds(rs, rows_per_sub)])
        plsc.subcore_barrier()
        extra = nchunks - (nchunks // _NS) * _NS
        nit = (nchunks // _NS) + jnp.where(sid < extra, 1, 0)

        def body(j, carry):
            base = cid * e_half + (sid + j * _NS) * _CH
            pltpu.sync_copy(src_h.at[pl.ds(base, _CH)], idx_b)
            pltpu.sync_copy(mo_h.at[pl.ds(base, _CH)], m_b)
            pltpu.sync_copy(m_b, agg_sh.at[idx_b], add=True)
            return carry

        lax.fori_loop(0, nit, body, 0)
        plsc.subcore_barrier()
        pltpu.sync_copy(agg_sh.at[pl.ds(rs, rows_per_sub)],
                        out_h.at[cid, pl.ds(rs, rows_per_sub)])

    return sk(mo, src, zeros_n)


def kernel(h_feats, x, edge_index, spatial_attr, positional_attr,
           W_single, W_spatial, W_pos,
           ew1, eb1, ew2, eb2, aw, ab, nw1, nb1, nw2, nb2):
    n, _ = h_feats.shape
    e = edge_index.shape[1]
    nd = W_single.shape[1]
    depth = ew1.shape[0]
    src = edge_index[0]
    dst = edge_index[1]
    xcols = [x[:, 0], x[:, 1], x[:, 2]]
    n_pad = ((n + 127) // 128) * 128
    zeros_n = jnp.zeros((n_pad, nd), _F32)

    h, p_tab, q_tab = _embed_call(h_feats, W_single,
                                  ew1[0, :nd], ew1[0, nd:2 * nd])

    psrc, qdst, rad = _gather_pq_sc(p_tab, q_tab, src, dst, xcols=xcols)
    rad = rad.reshape(e, 1)

    eattr = _eemb_call(spatial_attr, positional_attr, W_spatial, W_pos)

    for l in range(depth):
        if l > 0:
            psrc, qdst = _gather_pq_sc(p_tab, q_tab, src, dst)
        mo = _edge_call(
            psrc, qdst, eattr, rad,
            ew1[l, 2 * nd + 1:], ew1[l, 2 * nd].reshape(1, nd),
            eb1[l].reshape(1, nd), ew2[l], eb2[l].reshape(1, nd),
            aw[l].reshape(1, nd), ab[l].reshape(1, 1))
        aggp = _scatter_add_sc(mo, src, zeros_n)
        a0, a1 = aggp[0, :n], aggp[1, :n]
        if l + 1 < depth:
            h, p_tab, q_tab = _node_call(
                h, a0, a1, nw1[l, :nd], nw1[l, nd:],
                nb1[l].reshape(1, nd), nw2[l], nb2[l].reshape(1, nd),
                ew1[l + 1, :nd], ew1[l + 1, nd:2 * nd])
        else:
            h = _node_call(
                h, a0, a1, nw1[l, :nd], nw1[l, nd:],
                nb1[l].reshape(1, nd), nw2[l], nb2[l].reshape(1, nd))
    return h


# speedup vs baseline: 3.5757x; 1.1536x over previous
import functools

import jax
import jax.numpy as jnp
from jax import lax
from jax.experimental import pallas as pl
from jax.experimental.pallas import tpu as pltpu

try:
    from jax.experimental.pallas import tpu_sc as plsc
    _HAS_SC = True
except ImportError:
    plsc = None
    _HAS_SC = False

_F32 = jnp.float32

_NC = 2
_NS = 16
_NW = _NC * _NS
_CH = 128


def _node_block(n):
    for b in (400, 200, 100, 40, 8):
        if n % b == 0:
            return b
    return n


def _edge_block(e):
    for b in (2000, 1000, 400, 200, 40, 8):
        if e % b == 0:
            return b
    return e


def _embed_body(hf, ws, wa, wb, h_o, p_o, q_o):
    h = jnp.dot(hf[...], ws[...], preferred_element_type=_F32)
    h_o[...] = h
    p_o[...] = jnp.dot(h, wa[...], preferred_element_type=_F32)
    q_o[...] = jnp.dot(h, wb[...], preferred_element_type=_F32)


def _embed_call(h_feats, W_single, wa, wb):
    n, lm = h_feats.shape
    nd = W_single.shape[1]
    bn = _node_block(n)
    grid = (n // bn,)
    full = lambda s: pl.BlockSpec(s, lambda i: (0, 0))
    row = lambda c: pl.BlockSpec((bn, c), lambda i: (i, 0))
    out_sd = jax.ShapeDtypeStruct((n, nd), _F32)
    return pl.pallas_call(
        _embed_body,
        grid=grid,
        in_specs=[row(lm), full((lm, nd)), full((nd, nd)), full((nd, nd))],
        out_specs=[row(nd), row(nd), row(nd)],
        out_shape=[out_sd, out_sd, out_sd],
    )(h_feats, W_single, wa, wb)


def _eemb_body(sp, po, wsp, wpo, ea_o):
    ea = jnp.dot(sp[...], wsp[...], preferred_element_type=_F32)
    ea += jnp.dot(po[...], wpo[...], preferred_element_type=_F32)
    ea_o[...] = ea


def _eemb_call(spatial, pos, wsp, wpo):
    e, spd = spatial.shape
    ped = pos.shape[1]
    ed = wsp.shape[1]
    be = _edge_block(e)
    grid = (e // be,)
    full = lambda s: pl.BlockSpec(s, lambda i: (0, 0))
    row = lambda c: pl.BlockSpec((be, c), lambda i: (i, 0))
    return pl.pallas_call(
        _eemb_body,
        grid=grid,
        in_specs=[row(spd), row(ped), full((spd, ed)), full((ped, ed))],
        out_specs=row(ed),
        out_shape=jax.ShapeDtypeStruct((e, ed), _F32),
    )(spatial, pos, wsp, wpo)


def _edge_body(ps, qd, ea, rad, w1e, w1r, b1, w2, b2, awv, abv, mo_o):
    t = ps[...] + qd[...] + rad[...] * w1r[...] + b1[...]
    t += jnp.dot(ea[...], w1e[...], preferred_element_type=_F32)
    t = t * jax.nn.sigmoid(t)
    m = jnp.dot(t, w2[...], preferred_element_type=_F32) + b2[...]
    m = m * jax.nn.sigmoid(m)
    lg = jnp.sum(m * awv[...], axis=1, keepdims=True) + abv[...]
    mo_o[...] = m * jax.nn.sigmoid(lg)


def _edge_call(psrc, qdst, eattr, rad, w1e, w1r, b1, w2, b2, awv, abv):
    e, nd = psrc.shape
    ed = eattr.shape[1]
    be = _edge_block(e)
    grid = (e // be,)
    full = lambda s: pl.BlockSpec(s, lambda i: tuple(0 for _ in s))
    row = lambda c: pl.BlockSpec((be, c), lambda i: (i, 0))
    return pl.pallas_call(
        _edge_body,
        grid=grid,
        in_specs=[row(nd), row(nd), row(ed), row(1), full((ed, nd)),
                  full((1, nd)), full((1, nd)), full((nd, nd)),
                  full((1, nd)), full((1, nd)), full((1, 1))],
        out_specs=row(nd),
        out_shape=jax.ShapeDtypeStruct((e, nd), _F32),
    )(psrc, qdst, eattr, rad, w1e, w1r, b1, w2, b2, awv, abv)


def _node_body_pq(h, a0, a1, w1h, w1a, b1, w2, b2, wan, wbn, h_o, p_o, q_o):
    agg = a0[...] + a1[...]
    o = jnp.dot(h[...], w1h[...], preferred_element_type=_F32)
    o += jnp.dot(agg, w1a[...], preferred_element_type=_F32) + b1[...]
    o = o * jax.nn.sigmoid(o)
    o = jnp.dot(o, w2[...], preferred_element_type=_F32) + b2[...]
    hn = h[...] + o
    h_o[...] = hn
    p_o[...] = jnp.dot(hn, wan[...], preferred_element_type=_F32)
    q_o[...] = jnp.dot(hn, wbn[...], preferred_element_type=_F32)


def _node_body_last(h, a0, a1, w1h, w1a, b1, w2, b2, h_o):
    agg = a0[...] + a1[...]
    o = jnp.dot(h[...], w1h[...], preferred_element_type=_F32)
    o += jnp.dot(agg, w1a[...], preferred_element_type=_F32) + b1[...]
    o = o * jax.nn.sigmoid(o)
    o = jnp.dot(o, w2[...], preferred_element_type=_F32) + b2[...]
    h_o[...] = h[...] + o


def _node_call(h, a0, a1, w1h, w1a, b1, w2, b2, wan=None, wbn=None):
    n, nd = h.shape
    bn = _node_block(n)
    grid = (n // bn,)
    full = lambda s: pl.BlockSpec(s, lambda i: (0, 0))
    row = pl.BlockSpec((bn, nd), lambda i: (i, 0))
    out_sd = jax.ShapeDtypeStruct((n, nd), _F32)
    wspecs = [full((nd, nd)), full((nd, nd)), full((1, nd)), full((nd, nd)),
              full((1, nd))]
    if wan is None:
        return pl.pallas_call(
            _node_body_last,
            grid=grid,
            in_specs=[row, row, row] + wspecs,
            out_specs=row,
            out_shape=out_sd,
        )(h, a0, a1, w1h, w1a, b1, w2, b2)
    return pl.pallas_call(
        _node_body_pq,
        grid=grid,
        in_specs=[row, row, row] + wspecs + [full((nd, nd)), full((nd, nd))],
        out_specs=[row, row, row],
        out_shape=[out_sd, out_sd, out_sd],
    )(h, a0, a1, w1h, w1a, b1, w2, b2, wan, wbn)


_NB = 3


def _gather_pq_sc(p_tab, q_tab, src, dst, xcols=None):
    n, nd = p_tab.shape
    e = src.shape[0]
    nchunks = e // _CH
    with_rad = xcols is not None
    per_tile = -(-nchunks // _NW)
    nslots = -(-per_tile // _NB) * _NB
    kout = nslots // _NB
    mesh = plsc.VectorSubcoreMesh(core_axis_name="c", subcore_axis_name="s")
    out_type = [jax.ShapeDtypeStruct((e, nd), _F32),
                jax.ShapeDtypeStruct((e, nd), _F32)]
    scratch = ([pltpu.VMEM((_CH,), jnp.int32)] * (2 * _NB)
               + [pltpu.VMEM((_CH, nd), _F32)] * (2 * _NB))
    if with_rad:
        out_type.append(jax.ShapeDtypeStruct((e,), _F32))
        scratch += [pltpu.VMEM((n,), _F32)] * 3
        scratch += [pltpu.VMEM((_CH,), _F32)] * _NB
    scratch += [pltpu.SemaphoreType.DMA] * (2 * _NB)

    @functools.partial(
        pl.kernel, mesh=mesh, out_type=out_type, scratch_types=scratch,
        compiler_params=pltpu.CompilerParams(needs_layout_passes=False))
    def gk(*refs):
        it = iter(refs)
        p_h, q_h = next(it), next(it)
        xc_h = [next(it) for _ in range(3)] if with_rad else []
        src_h, dst_h = next(it), next(it)
        ps_o, qd_o = next(it), next(it)
        rad_o = next(it) if with_rad else None
        idx_s = [next(it) for _ in range(_NB)]
        idx_d = [next(it) for _ in range(_NB)]
        pbuf = [next(it) for _ in range(_NB)]
        qbuf = [next(it) for _ in range(_NB)]
        xv = [next(it) for _ in range(3)] if with_rad else []
        rbuf = [next(it) for _ in range(_NB)] if with_rad else []
        semg = [next(it) for _ in range(_NB)]
        semw = [next(it) for _ in range(_NB)]
        wid = lax.axis_index("s") * _NC + lax.axis_index("c")
        if with_rad:
            for k in range(3):
                pltpu.sync_copy(xc_h[k], xv[k])

        def cbase(j):
            c = wid + j * _NW
            return jnp.where(c < nchunks, c, wid) * _CH

        def fire_slot(j, b):
            base = cbase(j)
            pltpu.sync_copy(src_h.at[pl.ds(base, _CH)], idx_s[b])
            pltpu.sync_copy(dst_h.at[pl.ds(base, _CH)], idx_d[b])
            pltpu.async_copy(p_h.at[idx_s[b]], pbuf[b], semg[b])
            pltpu.async_copy(q_h.at[idx_d[b]], qbuf[b], semg[b])
            if with_rad:
                for i in range(_CH // 16):
                    s_ids = idx_s[b][pl.ds(i * 16, 16)]
                    d_ids = idx_d[b][pl.ds(i * 16, 16)]
                    acc = None
                    for k in range(3):
                        dd = (plsc.load_gather(xv[k], [s_ids])
                              - plsc.load_gather(xv[k], [d_ids]))
                        acc = dd * dd if acc is None else acc + dd * dd
                    rbuf[b][pl.ds(i * 16, 16)] = acc

        def wait_g(b):
            pltpu.make_async_copy(p_h.at[idx_s[b]], pbuf[b], semg[b]).wait()
            pltpu.make_async_copy(q_h.at[idx_d[b]], qbuf[b], semg[b]).wait()

        def fire_wb(j, b):
            base = cbase(j)
            pltpu.async_copy(pbuf[b], ps_o.at[pl.ds(base, _CH)], semw[b])
            pltpu.async_copy(qbuf[b], qd_o.at[pl.ds(base, _CH)], semw[b])
            if with_rad:
                pltpu.async_copy(rbuf[b], rad_o.at[pl.ds(base, _CH)], semw[b])

        def wait_wb(b):
            pltpu.make_async_copy(pbuf[b], ps_o.at[pl.ds(0, _CH)],
                                  semw[b]).wait()
            pltpu.make_async_copy(qbuf[b], qd_o.at[pl.ds(0, _CH)],
                                  semw[b]).wait()
            if with_rad:
                pltpu.make_async_copy(rbuf[b], rad_o.at[pl.ds(0, _CH)],
                                      semw[b]).wait()

        def outer(k, carry):
            for b in range(_NB):
                j = k * _NB + b

                @pl.when(k > 0)
                def _retire_wb(b=b):
                    wait_wb(b)

                fire_slot(j, b)
                bp = (b - 1) % _NB
                if b > 0:
                    wait_g(bp)
                    fire_wb(j - 1, bp)
                else:
                    @pl.when(k > 0)
                    def _retire_g(bp=bp, j=j):
                        wait_g(bp)
                        fire_wb(j - 1, bp)
            return carry

        lax.fori_loop(0, kout, outer, 0)
        last = nslots - 1
        bl = last % _NB
        wait_g(bl)
        fire_wb(last, bl)
        for b in range(_NB):
            wait_wb(b)

    if with_rad:
        return gk(p_tab, q_tab, *xcols, src, dst)
    return gk(p_tab, q_tab, src, dst)


def _scatter_add_sc(mo, src, zeros_n):
    e, nd = mo.shape
    n = zeros_n.shape[0]
    e_half = e // _NC
    nchunks = e_half // _CH
    rows_per_sub = n // _NS
    mesh = plsc.VectorSubcoreMesh(core_axis_name="c", subcore_axis_name="s")

    per_sub = -(-nchunks // _NS)
    nslots = -(-per_sub // _NB) * _NB
    kout = nslots // _NB

    @functools.partial(
        pl.kernel, mesh=mesh,
        out_type=jax.ShapeDtypeStruct((_NC, n, nd), _F32),
        scratch_types=(
            [pltpu.VMEM((_CH,), jnp.int32)] * _NB
            + [pltpu.VMEM((_CH, nd), _F32)] * _NB
            + [pltpu.VMEM_SHARED((n, nd), _F32)]
            + [pltpu.SemaphoreType.DMA] * _NB
        ))
    def sk(*refs):
        it = iter(refs)
        mo_h, src_h, z_h, out_h = next(it), next(it), next(it), next(it)
        idx = [next(it) for _ in range(_NB)]
        mb = [next(it) for _ in range(_NB)]
        agg_sh = next(it)
        seml = [next(it) for _ in range(_NB)]
        cid = lax.axis_index("c")
        sid = lax.axis_index("s")
        rs = sid * rows_per_sub
        pltpu.sync_copy(z_h.at[pl.ds(rs, rows_per_sub)],
                        agg_sh.at[pl.ds(rs, rows_per_sub)])
        plsc.subcore_barrier()

        def cond(j):
            return (sid + j * _NS) < nchunks

        def base(j):
            return cid * e_half + (sid + j * _NS) * _CH

        def fire_load(j, b):
            @pl.when(cond(j))
            def _(j=j, b=b):
                pltpu.async_copy(src_h.at[pl.ds(base(j), _CH)], idx[b],
                                 seml[b])
                pltpu.async_copy(mo_h.at[pl.ds(base(j), _CH)], mb[b],
                                 seml[b])

        def drain_scatter(j, b):
            @pl.when(cond(j))
            def _(j=j, b=b):
                pltpu.make_async_copy(src_h.at[pl.ds(0, _CH)], idx[b],
                                      seml[b]).wait()
                pltpu.make_async_copy(mo_h.at[pl.ds(0, _CH)], mb[b],
                                      seml[b]).wait()
                pltpu.sync_copy(mb[b], agg_sh.at[idx[b]], add=True)

        def outer(k, carry):
            for b in range(_NB):
                j = k * _NB + b
                fire_load(j, b)
                if b > 0:
                    drain_scatter(j - 1, b - 1)
                else:
                    @pl.when(k > 0)
                    def _(j=j):
                        drain_scatter(j - 1, _NB - 1)
            return carry

        lax.fori_loop(0, kout, outer, 0)
        drain_scatter(nslots - 1, (nslots - 1) % _NB)
        plsc.subcore_barrier()
        pltpu.sync_copy(agg_sh.at[pl.ds(rs, rows_per_sub)],
                        out_h.at[cid, pl.ds(rs, rows_per_sub)])

    return sk(mo, src, zeros_n)


def kernel(h_feats, x, edge_index, spatial_attr, positional_attr,
           W_single, W_spatial, W_pos,
           ew1, eb1, ew2, eb2, aw, ab, nw1, nb1, nw2, nb2):
    n, _ = h_feats.shape
    e = edge_index.shape[1]
    nd = W_single.shape[1]
    depth = ew1.shape[0]
    src = edge_index[0]
    dst = edge_index[1]
    xcols = [x[:, 0], x[:, 1], x[:, 2]]
    n_pad = ((n + 127) // 128) * 128
    zeros_n = jnp.zeros((n_pad, nd), _F32)

    h, p_tab, q_tab = _embed_call(h_feats, W_single,
                                  ew1[0, :nd], ew1[0, nd:2 * nd])

    psrc, qdst, rad = _gather_pq_sc(p_tab, q_tab, src, dst, xcols=xcols)
    rad = rad.reshape(e, 1)

    eattr = _eemb_call(spatial_attr, positional_attr, W_spatial, W_pos)

    for l in range(depth):
        if l > 0:
            psrc, qdst = _gather_pq_sc(p_tab, q_tab, src, dst)
        mo = _edge_call(
            psrc, qdst, eattr, rad,
            ew1[l, 2 * nd + 1:], ew1[l, 2 * nd].reshape(1, nd),
            eb1[l].reshape(1, nd), ew2[l], eb2[l].reshape(1, nd),
            aw[l].reshape(1, nd), ab[l].reshape(1, 1))
        aggp = _scatter_add_sc(mo, src, zeros_n)
        a0, a1 = aggp[0, :n], aggp[1, :n]
        if l + 1 < depth:
            h, p_tab, q_tab = _node_call(
                h, a0, a1, nw1[l, :nd], nw1[l, nd:],
                nb1[l].reshape(1, nd), nw2[l], nb2[l].reshape(1, nd),
                ew1[l + 1, :nd], ew1[l + 1, nd:2 * nd])
        else:
            h = _node_call(
                h, a0, a1, nw1[l, :nd], nw1[l, nd:],
                nb1[l].reshape(1, nd), nw2[l], nb2[l].reshape(1, nd))
    return h
